# baseline (speedup 1.0000x reference)
import jax
import jax.numpy as jnp
from jax import lax
from jax.experimental import pallas as pl
from jax.experimental.pallas import tpu as pltpu
from jax.experimental.pallas import tpu_sc as plsc

NUM_CORES = 2
NUM_SUBCORES = 16
LANES = 16
NUM_WORKERS = NUM_CORES * NUM_SUBCORES
BATCH = 16384
BPW = BATCH // NUM_WORKERS
DIM = 16
NROWS = 1000000
CHUNK = 16
NCHUNKS = BPW // CHUNK


def _sc_kernel(uidx_hbm, iidx_hbm, ut_hbm, it_hbm, w_hbm, out_hbm,
               uidx_v, iidx_v, ublk_v, iblk_v, out_v, w_v, sem):
    wid = lax.axis_index("s") * NUM_CORES + lax.axis_index("c")
    base = pl.multiple_of(wid * BPW, BPW)

    pltpu.sync_copy(uidx_hbm.at[pl.ds(base, BPW)], uidx_v)
    pltpu.sync_copy(iidx_hbm.at[pl.ds(base, BPW)], iidx_v)
    pltpu.sync_copy(w_hbm, w_v)

    uv3 = ut_hbm.reshape(2, 8, NROWS)
    iv3 = it_hbm.reshape(2, 8, NROWS)

    wvec = w_v[...]
    lanes = lax.iota(jnp.int32, LANES)
    _dnums = lax.GatherDimensionNumbers(
        offset_dims=(), collapsed_slice_dims=(0,), start_index_map=(0,))

    def _bcast(x, d):
        return lax.gather(
            x, jnp.full((LANES, 1), d, jnp.int32), _dnums, slice_sizes=(1,),
            mode=lax.GatherScatterMode.PROMISE_IN_BOUNDS)

    wb = [_bcast(wvec, d) for d in range(DIM)]

    ublk2 = ublk_v.reshape(CHUNK * DIM, 128)
    iblk2 = iblk_v.reshape(CHUNK * DIM, 128)

    def chunk_body(c, carry):
        off = pl.multiple_of(c * CHUNK, CHUNK)
        uvec = uidx_v[pl.ds(off, CHUNK)]
        ivec = iidx_v[pl.ds(off, CHUNK)]
        ustart = (uvec >> 7) << 7
        istart = (ivec >> 7) << 7
        copies = []
        for j in range(CHUNK):
            uo = pl.multiple_of(ustart[j], 128)
            io = pl.multiple_of(istart[j], 128)
            for tr in range(2):
                copies.append(pltpu.async_copy(
                    uv3.at[tr, :, pl.ds(uo, 128)], ublk_v.at[j, tr], sem))
                copies.append(pltpu.async_copy(
                    iv3.at[tr, :, pl.ds(io, 128)], iblk_v.at[j, tr], sem))
        for cp in copies:
            cp.wait()

        um = uvec & 127
        im = ivec & 127
        row_base = lanes * DIM
        acc = None
        for d in range(DIM):
            ug = plsc.load_gather(ublk2, [row_base + d, um])
            ig = plsc.load_gather(iblk2, [row_base + d, im])
            term = ug * ig * wb[d]
            acc = term if acc is None else acc + term
        out_v[pl.ds(off, CHUNK)] = 1.0 / (1.0 + jnp.exp(-acc))
        return carry

    lax.fori_loop(0, NCHUNKS, chunk_body, 0)

    pltpu.sync_copy(out_v, out_hbm.at[pl.ds(base, BPW)])


@jax.jit
def _run(user_indices, item_indices, emb_user_t, emb_item_t, w_flat):
    mesh = plsc.VectorSubcoreMesh(core_axis_name="c", subcore_axis_name="s")
    return pl.kernel(
        _sc_kernel,
        out_type=jax.ShapeDtypeStruct((BATCH,), jnp.float32),
        mesh=mesh,
        compiler_params=pltpu.CompilerParams(needs_layout_passes=False),
        scratch_types=[
            pltpu.VMEM((BPW,), jnp.int32),
            pltpu.VMEM((BPW,), jnp.int32),
            pltpu.VMEM((CHUNK, 2, 8, 128), jnp.float32),
            pltpu.VMEM((CHUNK, 2, 8, 128), jnp.float32),
            pltpu.VMEM((BPW,), jnp.float32),
            pltpu.VMEM((LANES,), jnp.float32),
            pltpu.SemaphoreType.DMA,
        ],
    )(user_indices, item_indices, emb_user_t, emb_item_t, w_flat)


def kernel(user_indices, item_indices, emb_user, emb_item, w_gmf):
    return _run(
        user_indices.astype(jnp.int32),
        item_indices.astype(jnp.int32),
        emb_user.T,
        emb_item.T,
        w_gmf.reshape(DIM),
    )


# speedup vs baseline: 1.1696x; 1.1696x over previous
import jax
import jax.numpy as jnp
from jax import lax
from jax.experimental import pallas as pl
from jax.experimental.pallas import tpu as pltpu
from jax.experimental.pallas import tpu_sc as plsc

NUM_CORES = 2
NUM_SUBCORES = 16
LANES = 16
NUM_WORKERS = NUM_CORES * NUM_SUBCORES
BATCH = 16384
BPW = BATCH // NUM_WORKERS
DIM = 16
NROWS = 1000000
CHUNK = 8
NCHUNKS = BPW // CHUNK
PAD = LANES


def _sc_kernel(uidx_hbm, iidx_hbm, ut_hbm, it_hbm, w_hbm, out_hbm,
               uidx_v, iidx_v, ublkA, iblkA, ublkB, iblkB, out_v, w_v, sem):
    wid = lax.axis_index("s") * NUM_CORES + lax.axis_index("c")
    base = pl.multiple_of(wid * BPW, BPW)

    pltpu.sync_copy(uidx_hbm.at[pl.ds(base, BPW)], uidx_v.at[pl.ds(0, BPW)])
    pltpu.sync_copy(iidx_hbm.at[pl.ds(base, BPW)], iidx_v.at[pl.ds(0, BPW)])
    pltpu.sync_copy(w_hbm, w_v)

    uv3 = ut_hbm.reshape(2, 8, NROWS)
    iv3 = it_hbm.reshape(2, 8, NROWS)

    wvec = w_v[...]
    lanes = lax.iota(jnp.int32, LANES)
    rowsel = (lanes & (CHUNK - 1)) * LANES
    _dnums = lax.GatherDimensionNumbers(
        offset_dims=(), collapsed_slice_dims=(0,), start_index_map=(0,))

    def _bcast(x, d):
        return lax.gather(
            x, jnp.full((LANES, 1), d, jnp.int32), _dnums, slice_sizes=(1,),
            mode=lax.GatherScatterMode.PROMISE_IN_BOUNDS)

    wb = [_bcast(wvec, d) for d in range(DIM)]

    def issue(c, ublk, iblk):
        off = pl.multiple_of(c * CHUNK, CHUNK)
        ustart = (uidx_v[pl.ds(off, LANES)] >> 7) << 7
        istart = (iidx_v[pl.ds(off, LANES)] >> 7) << 7
        for j in range(CHUNK):
            uo = pl.multiple_of(ustart[j], 128)
            io = pl.multiple_of(istart[j], 128)
            pltpu.async_copy(uv3.at[:, :, pl.ds(uo, 128)], ublk.at[j], sem)
            pltpu.async_copy(iv3.at[:, :, pl.ds(io, 128)], iblk.at[j], sem)

    def drain(ublk, iblk):
        for j in range(CHUNK):
            pltpu.make_async_copy(
                uv3.at[:, :, pl.ds(0, 128)], ublk.at[j], sem).wait()
            pltpu.make_async_copy(
                iv3.at[:, :, pl.ds(0, 128)], iblk.at[j], sem).wait()

    def process(c, ublk, iblk):
        off = pl.multiple_of(c * CHUNK, CHUNK)
        um = uidx_v[pl.ds(off, LANES)] & 127
        im = iidx_v[pl.ds(off, LANES)] & 127
        ub2 = ublk.reshape(CHUNK * DIM, 128)
        ib2 = iblk.reshape(CHUNK * DIM, 128)
        acc = None
        for d in range(DIM):
            ug = plsc.load_gather(ub2, [rowsel + d, um])
            ig = plsc.load_gather(ib2, [rowsel + d, im])
            term = ug * ig * wb[d]
            acc = term if acc is None else acc + term
        out_v[pl.ds(off, LANES)] = 1.0 / (1.0 + jnp.exp(-acc))

    issue(0, ublkA, iblkA)
    issue(1, ublkB, iblkB)

    def body(cc, carry):
        c0 = cc * 2
        c1 = c0 + 1
        drain(ublkA, iblkA)
        process(c0, ublkA, iblkA)
        issue((c0 + 2) & (NCHUNKS - 1), ublkA, iblkA)
        drain(ublkB, iblkB)
        process(c1, ublkB, iblkB)
        issue((c1 + 2) & (NCHUNKS - 1), ublkB, iblkB)
        return carry

    lax.fori_loop(0, NCHUNKS // 2, body, 0)
    drain(ublkA, iblkA)
    drain(ublkB, iblkB)

    pltpu.sync_copy(out_v.at[pl.ds(0, BPW)], out_hbm.at[pl.ds(base, BPW)])


@jax.jit
def _run(user_indices, item_indices, emb_user_t, emb_item_t, w_flat):
    mesh = plsc.VectorSubcoreMesh(core_axis_name="c", subcore_axis_name="s")
    return pl.kernel(
        _sc_kernel,
        out_type=jax.ShapeDtypeStruct((BATCH,), jnp.float32),
        mesh=mesh,
        compiler_params=pltpu.CompilerParams(needs_layout_passes=False),
        scratch_types=[
            pltpu.VMEM((BPW + PAD,), jnp.int32),
            pltpu.VMEM((BPW + PAD,), jnp.int32),
            pltpu.VMEM((CHUNK, 2, 8, 128), jnp.float32),
            pltpu.VMEM((CHUNK, 2, 8, 128), jnp.float32),
            pltpu.VMEM((CHUNK, 2, 8, 128), jnp.float32),
            pltpu.VMEM((CHUNK, 2, 8, 128), jnp.float32),
            pltpu.VMEM((BPW + PAD,), jnp.float32),
            pltpu.VMEM((LANES,), jnp.float32),
            pltpu.SemaphoreType.DMA,
        ],
    )(user_indices, item_indices, emb_user_t, emb_item_t, w_flat)


def kernel(user_indices, item_indices, emb_user, emb_item, w_gmf):
    return _run(
        user_indices.astype(jnp.int32),
        item_indices.astype(jnp.int32),
        emb_user.T,
        emb_item.T,
        w_gmf.reshape(DIM),
    )
